# baseline (speedup 1.0000x reference)
import jax
import jax.numpy as jnp
from jax.experimental import pallas as pl
from jax.experimental.pallas import tpu as pltpu

_N = 4096
_CHUNKS = 8
_ROWS_PER_CHUNK = _N // _CHUNKS


def _dma_body(in_ref, out_ref, sem):
    for c in range(_CHUNKS):
        rows = pl.ds(c * _ROWS_PER_CHUNK, _ROWS_PER_CHUNK)
        pltpu.make_async_copy(in_ref.at[rows], out_ref.at[rows], sem).start()
    for c in range(_CHUNKS):
        rows = pl.ds(c * _ROWS_PER_CHUNK, _ROWS_PER_CHUNK)
        pltpu.make_async_copy(in_ref.at[rows], out_ref.at[rows], sem).wait()


def kernel(inputs, w):
    del inputs
    x = jnp.transpose(w, (0, 2, 1))
    y = pl.pallas_call(
        _dma_body,
        in_specs=[pl.BlockSpec(memory_space=pl.ANY)],
        out_specs=pl.BlockSpec(memory_space=pl.ANY),
        out_shape=jax.ShapeDtypeStruct((_N, 2, _N), jnp.float32),
        scratch_shapes=[pltpu.SemaphoreType.DMA],
    )(x)
    return jnp.transpose(y, (0, 2, 1))


# speedup vs baseline: 49.0590x; 49.0590x over previous
import jax
import jax.numpy as jnp
from jax.experimental import pallas as pl
from jax.experimental.pallas import tpu as pltpu

_N = 4096
_BLOCK_ROWS = 256


def _copy_body(in_ref, out_ref):
    out_ref[...] = in_ref[...]


def kernel(inputs, w):
    del inputs
    x = jnp.transpose(w, (0, 2, 1))
    y = pl.pallas_call(
        _copy_body,
        grid=(_N // _BLOCK_ROWS,),
        in_specs=[pl.BlockSpec((_BLOCK_ROWS, 2, _N), lambda i: (i, 0, 0))],
        out_specs=pl.BlockSpec((_BLOCK_ROWS, 2, _N), lambda i: (i, 0, 0)),
        out_shape=jax.ShapeDtypeStruct((_N, 2, _N), jnp.float32),
    )(x)
    return jnp.transpose(y, (0, 2, 1))
